# baseline (speedup 1.0000x reference)
import functools

import jax
import jax.numpy as jnp
from jax import lax
from jax.experimental import pallas as pl
from jax.experimental.pallas import tpu as pltpu
from jax.experimental.pallas import tpu_sc as plsc

_VOCAB = 1000000
_EMB = 64
_CTX = 200
_BLK = 16384
_NB = (_VOCAB + _BLK - 1) // _BLK

_NA = 104
_NionB = _CTX - _NA


def _sc_body(idx_hbm, tab_hbm, out_hbm, ia, ib, ra, rb, acc, s1, s2):
    wid = lax.axis_index("s") * 2 + lax.axis_index("c")

    @pl.when(wid == 0)
    def _():
        pltpu.sync_copy(idx_hbm.at[pl.ds(0, _NA)], ia)
        pltpu.sync_copy(idx_hbm.at[pl.ds(_NA, _NionB)], ib)
        ca = pltpu.async_copy(tab_hbm.at[ia], ra, s1)
        cb = pltpu.async_copy(tab_hbm.at[ib], rb, s2)
        ca.wait()
        cb.wait()

        def accum(rows_ref, m, init):
            def body(i, carry):
                return tuple(
                    carry[j] + rows_ref[i, pl.ds(j * 16, 16)] for j in range(4)
                )

            return lax.fori_loop(0, m, body, init)

        z = tuple(jnp.zeros((16,), jnp.float32) for _ in range(4))
        z = accum(ra, _NA, z)
        z = accum(rb, _NionB, z)
        for j in range(4):
            acc[pl.ds(j * 16, 16)] = z[j] * (1.0 / (2.0 * 100.0))
        pltpu.sync_copy(acc, out_hbm)


def _sc_gather_sum(idx, table):
    mesh = plsc.VectorSubcoreMesh(core_axis_name="c", subcore_axis_name="s")
    k = functools.partial(
        pl.kernel,
        out_type=jax.ShapeDtypeStruct((_EMB,), jnp.float32),
        mesh=mesh,
        compiler_params=pltpu.CompilerParams(use_tc_tiling_on_sc=False),
        scratch_types=[
            pltpu.VMEM((_NA,), jnp.int32),
            pltpu.VMEM((_NionB,), jnp.int32),
            pltpu.VMEM((_NA, _EMB), jnp.float32),
            pltpu.VMEM((_NionB, _EMB), jnp.float32),
            pltpu.VMEM((_EMB,), jnp.float32),
            pltpu.SemaphoreType.DMA,
            pltpu.SemaphoreType.DMA,
        ],
    )(_sc_body)
    return k(idx, table)


def _mv_body(v_ref, w_ref, lg_ref, bm_ref, bs_ref):
    i = pl.program_id(0)
    blk = lax.dot_general(
        v_ref[...],
        w_ref[...],
        (((1,), (1,)), ((), ())),
        preferred_element_type=jnp.float32,
    )
    lg_ref[...] = blk
    limit = _VOCAB - i * _BLK
    cols = lax.broadcasted_iota(jnp.int32, (1, _BLK), 1)
    mblk = jnp.where(cols < limit, blk, -1e30)
    bm = jnp.max(mblk, axis=1, keepdims=True)
    bs = jnp.sum(jnp.exp(mblk - bm), axis=1, keepdims=True)
    bm_ref[...] = jnp.broadcast_to(bm, (1, 128))
    bs_ref[...] = jnp.broadcast_to(bs, (1, 128))


def _norm_body(lg_ref, bm_ref, bs_ref, out_ref):
    gm = jnp.max(bm_ref[...], axis=1, keepdims=True)
    t = bs_ref[...] * jnp.exp(bm_ref[...] - gm)
    zz = jnp.sum(t, axis=1, keepdims=True) * (1.0 / 128.0)
    out_ref[...] = lg_ref[...] - (gm + jnp.log(zz))


def _tc_logits(v2, out_weight):
    logits, bm, bs = pl.pallas_call(
        _mv_body,
        grid=(_NB,),
        in_specs=[
            pl.BlockSpec((1, _EMB), lambda i: (0, 0)),
            pl.BlockSpec((_BLK, _EMB), lambda i: (i, 0)),
        ],
        out_specs=[
            pl.BlockSpec((1, _BLK), lambda i: (0, i)),
            pl.BlockSpec((1, 128), lambda i: (0, i)),
            pl.BlockSpec((1, 128), lambda i: (0, i)),
        ],
        out_shape=[
            jax.ShapeDtypeStruct((1, _NB * _BLK), jnp.float32),
            jax.ShapeDtypeStruct((1, _NB * 128), jnp.float32),
            jax.ShapeDtypeStruct((1, _NB * 128), jnp.float32),
        ],
    )(v2, out_weight)
    out = pl.pallas_call(
        _norm_body,
        grid=(_NB,),
        in_specs=[
            pl.BlockSpec((1, _BLK), lambda i: (0, i)),
            pl.BlockSpec((1, _NB * 128), lambda i: (0, 0)),
            pl.BlockSpec((1, _NB * 128), lambda i: (0, 0)),
        ],
        out_specs=pl.BlockSpec((1, _BLK), lambda i: (0, i)),
        out_shape=jax.ShapeDtypeStruct((1, _NB * _BLK), jnp.float32),
    )(logits, bm, bs)
    return out[:, :_VOCAB]


def kernel(input, embedding_weight, out_weight):
    idx = input.astype(jnp.int32)
    v = _sc_gather_sum(idx, embedding_weight)
    return _tc_logits(v.reshape(1, _EMB), out_weight)


# speedup vs baseline: 1.2807x; 1.2807x over previous
import functools

import jax
import jax.numpy as jnp
from jax import lax
from jax.experimental import pallas as pl
from jax.experimental.pallas import tpu as pltpu
from jax.experimental.pallas import tpu_sc as plsc

_VOCAB = 1000000
_EMB = 64
_CTX = 200
_HALF = _VOCAB // 2
_BLK = 16384
_NB = (_HALF + _BLK - 1) // _BLK

_NA = 104
_NBC = _CTX - _NA


def _sc_body(idx_hbm, tab_hbm, out_hbm, ia, ib, ra, rb, acc, s1, s2):
    wid = lax.axis_index("s") * 2 + lax.axis_index("c")

    @pl.when(wid == 0)
    def _():
        pltpu.sync_copy(idx_hbm.at[pl.ds(0, _NA)], ia)
        pltpu.sync_copy(idx_hbm.at[pl.ds(_NA, _NBC)], ib)
        ca = pltpu.async_copy(tab_hbm.at[ia], ra, s1)
        cb = pltpu.async_copy(tab_hbm.at[ib], rb, s2)
        ca.wait()
        cb.wait()

        def accum(rows_ref, m, init):
            def body(i, carry):
                return tuple(
                    carry[j] + rows_ref[i, pl.ds(j * 16, 16)] for j in range(4)
                )

            return lax.fori_loop(0, m, body, init)

        z = tuple(jnp.zeros((16,), jnp.float32) for _ in range(4))
        z = accum(ra, _NA, z)
        z = accum(rb, _NBC, z)
        for j in range(4):
            acc[pl.ds(j * 16, 16)] = z[j] * (1.0 / (2.0 * 100.0))
        pltpu.sync_copy(acc, out_hbm)


def _sc_gather_sum(idx, table):
    mesh = plsc.VectorSubcoreMesh(core_axis_name="c", subcore_axis_name="s")
    k = functools.partial(
        pl.kernel,
        out_type=jax.ShapeDtypeStruct((_EMB,), jnp.float32),
        mesh=mesh,
        compiler_params=pltpu.CompilerParams(use_tc_tiling_on_sc=False),
        scratch_types=[
            pltpu.VMEM((_NA,), jnp.int32),
            pltpu.VMEM((_NBC,), jnp.int32),
            pltpu.VMEM((_NA, _EMB), jnp.float32),
            pltpu.VMEM((_NBC, _EMB), jnp.float32),
            pltpu.VMEM((_EMB,), jnp.float32),
            pltpu.SemaphoreType.DMA,
            pltpu.SemaphoreType.DMA,
        ],
    )(_sc_body)
    return k(idx, table)


def _mv_body(vp_ref, w_ref, lg_ref, bm_ref, bs_ref):
    i = pl.program_id(0)
    z2 = lax.dot_general(
        vp_ref[...],
        w_ref[...],
        (((1,), (1,)), ((), ())),
        preferred_element_type=jnp.float32,
    )
    lg_ref[...] = z2
    del i
    bm_ref[...] = jnp.zeros((1, 128), jnp.float32)
    bs_ref[...] = jnp.ones((1, 128), jnp.float32)


def _norm_body(lg_ref, bm_ref, bs_ref, out_ref):
    gm = jnp.max(bm_ref[...], axis=1, keepdims=True)
    t = bs_ref[...] * jnp.exp(bm_ref[...] - gm)
    zz = jnp.sum(t, axis=1, keepdims=True) * (1.0 / 128.0)
    out_ref[...] = lg_ref[...] - (gm + jnp.log(zz))


def _tc_logits(vpair, w2):
    logits, bm, bs = pl.pallas_call(
        _mv_body,
        grid=(_NB,),
        in_specs=[
            pl.BlockSpec((2, 128), lambda i: (0, 0)),
            pl.BlockSpec((_BLK, 128), lambda i: (i, 0)),
        ],
        out_specs=[
            pl.BlockSpec((2, _BLK), lambda i: (0, i)),
            pl.BlockSpec((1, 128), lambda i: (0, i)),
            pl.BlockSpec((1, 128), lambda i: (0, i)),
        ],
        out_shape=[
            jax.ShapeDtypeStruct((2, _NB * _BLK), jnp.float32),
            jax.ShapeDtypeStruct((1, _NB * 128), jnp.float32),
            jax.ShapeDtypeStruct((1, _NB * 128), jnp.float32),
        ],
    )(vpair, w2)
    return logits
    out = pl.pallas_call(
        _norm_body,
        grid=(_NB,),
        in_specs=[
            pl.BlockSpec((2, _BLK), lambda i: (0, i)),
            pl.BlockSpec((1, _NB * 128), lambda i: (0, 0)),
            pl.BlockSpec((1, _NB * 128), lambda i: (0, 0)),
        ],
        out_specs=pl.BlockSpec((2, _BLK), lambda i: (0, i)),
        out_shape=jax.ShapeDtypeStruct((2, _NB * _BLK), jnp.float32),
    )(logits, bm, bs)
    return out


def kernel(input, embedding_weight, out_weight):
    idx = input.astype(jnp.int32)
    w2 = out_weight.reshape(_HALF, 128)
    v = jnp.sum(jnp.take(embedding_weight, idx, axis=0), axis=0) / 200.0
    zeros = jnp.zeros((_EMB,), jnp.float32)
    vpair = jnp.stack(
        [jnp.concatenate([v, zeros]), jnp.concatenate([zeros, v])]
    )
    z2 = _tc_logits(vpair, w2)
    return z2


# speedup vs baseline: 1.3502x; 1.0542x over previous
import functools

import jax
import jax.numpy as jnp
from jax import lax
from jax.experimental import pallas as pl
from jax.experimental.pallas import tpu as pltpu

_VOCAB = 1000000
_EMB = 64
_CTX = 200
_CH = 8000
_NC = _VOCAB // _CH
_NBUF = 8
_SCALE = 1.0 / (2.0 * 100.0)


def _mv_body(idx_ref, emb_hbm, w_hbm, lg_ref, bm_ref, bs_ref, wbuf, gbuf, vbuf, sems, gsem):
    i = pl.program_id(0)

    @pl.when(i == 0)
    def _prologue():
        for j in range(_NBUF):
            pltpu.make_async_copy(
                w_hbm.at[pl.ds(j * _CH, _CH)], wbuf.at[j], sems.at[j]
            ).start()
        for t in range(_CTX):
            pltpu.make_async_copy(
                emb_hbm.at[pl.ds(idx_ref[t], 1)], gbuf.at[pl.ds(t, 1)], gsem
            ).start()
        for t in range(_CTX):
            pltpu.make_async_copy(
                emb_hbm.at[pl.ds(0, 1)], gbuf.at[pl.ds(0, 1)], gsem
            ).wait()
        vbuf[...] = jnp.sum(gbuf[...], axis=0, keepdims=True) * _SCALE

    b = lax.rem(i, _NBUF)
    pltpu.make_async_copy(
        w_hbm.at[pl.ds(i * _CH, _CH)], wbuf.at[b], sems.at[b]
    ).wait()

    sub = _CH // 8
    pieces = []
    for r in range(8):
        pieces.append(
            lax.dot_general(
                vbuf[...],
                wbuf[b, pl.ds(r * sub, sub), :],
                (((1,), (1,)), ((), ())),
                preferred_element_type=jnp.float32,
            )
        )
    lg_ref[...] = jnp.concatenate(pieces, axis=0)
    bm = pieces[0].max(axis=1, keepdims=True)
    for r in range(1, 8):
        bm = jnp.maximum(bm, pieces[r].max(axis=1, keepdims=True))
    bs = jnp.zeros((1, 1), jnp.float32)
    for r in range(8):
        bs = bs + jnp.sum(jnp.exp(pieces[r] - bm), axis=1, keepdims=True)
    bm_ref[...] = jnp.broadcast_to(bm, (1, 128))
    bs_ref[...] = jnp.broadcast_to(bs, (1, 128))

    @pl.when(i + _NBUF < _NC)
    def _refill():
        pltpu.make_async_copy(
            w_hbm.at[pl.ds((i + _NBUF) * _CH, _CH)], wbuf.at[b], sems.at[b]
        ).start()


def _norm_body(lg_ref, bm_ref, bs_ref, out_ref):
    gm = jnp.max(bm_ref[...], axis=1, keepdims=True)
    t = bs_ref[...] * jnp.exp(bm_ref[...] - gm)
    zz = jnp.sum(t, axis=1, keepdims=True) * (1.0 / 128.0)
    out_ref[...] = lg_ref[...] - (gm + jnp.log(zz))


def _tc_logits(idx, emb, w):
    nrow = _NC * 8
    logits, bm, bs = pl.pallas_call(
        _mv_body,
        grid=(_NC,),
        in_specs=[
            pl.BlockSpec(memory_space=pltpu.SMEM),
            pl.BlockSpec(memory_space=pl.ANY),
            pl.BlockSpec(memory_space=pl.ANY),
        ],
        out_specs=[
            pl.BlockSpec((8, _CH // 8), lambda i: (i, 0)),
            pl.BlockSpec((1, 128), lambda i: (0, i)),
            pl.BlockSpec((1, 128), lambda i: (0, i)),
        ],
        out_shape=[
            jax.ShapeDtypeStruct((nrow, _CH // 8), jnp.float32),
            jax.ShapeDtypeStruct((1, _NC * 128), jnp.float32),
            jax.ShapeDtypeStruct((1, _NC * 128), jnp.float32),
        ],
        scratch_shapes=[
            pltpu.VMEM((_NBUF, _CH, _EMB), jnp.float32),
            pltpu.VMEM((_CTX, _EMB), jnp.float32),
            pltpu.VMEM((1, _EMB), jnp.float32),
            pltpu.SemaphoreType.DMA((_NBUF,)),
            pltpu.SemaphoreType.DMA,
        ],
    )(idx, emb, w)
    out = pl.pallas_call(
        _norm_body,
        grid=(25,),
        in_specs=[
            pl.BlockSpec((40, _CH // 8), lambda i: (i, 0)),
            pl.BlockSpec((1, _NC * 128), lambda i: (0, 0)),
            pl.BlockSpec((1, _NC * 128), lambda i: (0, 0)),
        ],
        out_specs=pl.BlockSpec((40, _CH // 8), lambda i: (i, 0)),
        out_shape=jax.ShapeDtypeStruct((nrow, _CH // 8), jnp.float32),
    )(logits, bm, bs)
    return out.reshape(1, _VOCAB)


def kernel(input, embedding_weight, out_weight):
    idx = input.astype(jnp.int32)
    return _tc_logits(idx, embedding_weight, out_weight)


# speedup vs baseline: 1.3508x; 1.0005x over previous
import functools

import jax
import jax.numpy as jnp
from jax import lax
from jax.experimental import pallas as pl
from jax.experimental.pallas import tpu as pltpu

_VOCAB = 1000000
_EMB = 64
_CTX = 200
_CH = 8000
_NC = _VOCAB // _CH
_NBUF = 8
_SCALE = 1.0 / (2.0 * 100.0)


def _mv_body(idx_ref, emb_hbm, w_hbm, lg_ref, bm_ref, bs_ref, wbuf, gbuf, vbuf, sems, gsem):
    i = pl.program_id(0)

    @pl.when(i == 0)
    def _prologue():
        for t in range(_CTX):
            pltpu.make_async_copy(
                emb_hbm.at[pl.ds(idx_ref[t], 1)], gbuf.at[pl.ds(t, 1)], gsem.at[t % 8]
            ).start()
        for j in range(_NBUF):
            pltpu.make_async_copy(
                w_hbm.at[pl.ds(j * _CH, _CH)], wbuf.at[j], sems.at[j]
            ).start()
        for t in range(_CTX):
            pltpu.make_async_copy(
                emb_hbm.at[pl.ds(0, 1)], gbuf.at[pl.ds(0, 1)], gsem.at[t % 8]
            ).wait()
        vbuf[...] = jnp.sum(gbuf[...], axis=0, keepdims=True) * _SCALE

    b = lax.rem(i, _NBUF)
    pltpu.make_async_copy(
        w_hbm.at[pl.ds(i * _CH, _CH)], wbuf.at[b], sems.at[b]
    ).wait()

    sub = _CH // 8
    pieces = []
    for r in range(8):
        pieces.append(
            lax.dot_general(
                vbuf[...],
                wbuf[b, pl.ds(r * sub, sub), :],
                (((1,), (1,)), ((), ())),
                preferred_element_type=jnp.float32,
            )
        )
    lg_ref[...] = jnp.concatenate(pieces, axis=0)
    bm = pieces[0].max(axis=1, keepdims=True)
    for r in range(1, 8):
        bm = jnp.maximum(bm, pieces[r].max(axis=1, keepdims=True))
    bs = jnp.zeros((1, 1), jnp.float32)
    for r in range(8):
        bs = bs + jnp.sum(jnp.exp(pieces[r] - bm), axis=1, keepdims=True)
    bm_ref[...] = jnp.broadcast_to(bm, (1, 128))
    bs_ref[...] = jnp.broadcast_to(bs, (1, 128))

    @pl.when(i + _NBUF < _NC)
    def _refill():
        pltpu.make_async_copy(
            w_hbm.at[pl.ds((i + _NBUF) * _CH, _CH)], wbuf.at[b], sems.at[b]
        ).start()


def _norm_body(lg_ref, bm_ref, bs_ref, out_ref):
    gm = jnp.max(bm_ref[...], axis=1, keepdims=True)
    t = bs_ref[...] * jnp.exp(bm_ref[...] - gm)
    zz = jnp.sum(t, axis=1, keepdims=True) * (1.0 / 128.0)
    out_ref[...] = lg_ref[...] - (gm + jnp.log(zz))


def _tc_logits(idx, emb, w):
    nrow = _NC * 8
    logits, bm, bs = pl.pallas_call(
        _mv_body,
        grid=(_NC,),
        in_specs=[
            pl.BlockSpec(memory_space=pltpu.SMEM),
            pl.BlockSpec(memory_space=pl.ANY),
            pl.BlockSpec(memory_space=pl.ANY),
        ],
        out_specs=[
            pl.BlockSpec((8, _CH // 8), lambda i: (i, 0)),
            pl.BlockSpec((1, 128), lambda i: (0, i)),
            pl.BlockSpec((1, 128), lambda i: (0, i)),
        ],
        out_shape=[
            jax.ShapeDtypeStruct((nrow, _CH // 8), jnp.float32),
            jax.ShapeDtypeStruct((1, _NC * 128), jnp.float32),
            jax.ShapeDtypeStruct((1, _NC * 128), jnp.float32),
        ],
        scratch_shapes=[
            pltpu.VMEM((_NBUF, _CH, _EMB), jnp.float32),
            pltpu.VMEM((_CTX, _EMB), jnp.float32),
            pltpu.VMEM((1, _EMB), jnp.float32),
            pltpu.SemaphoreType.DMA((_NBUF,)),
            pltpu.SemaphoreType.DMA((8,)),
        ],
    )(idx, emb, w)
    out = pl.pallas_call(
        _norm_body,
        grid=(25,),
        in_specs=[
            pl.BlockSpec((40, _CH // 8), lambda i: (i, 0)),
            pl.BlockSpec((1, _NC * 128), lambda i: (0, 0)),
            pl.BlockSpec((1, _NC * 128), lambda i: (0, 0)),
        ],
        out_specs=pl.BlockSpec((40, _CH // 8), lambda i: (i, 0)),
        out_shape=jax.ShapeDtypeStruct((nrow, _CH // 8), jnp.float32),
    )(logits, bm, bs)
    return out.reshape(1, _VOCAB)


def kernel(input, embedding_weight, out_weight):
    idx = input.astype(jnp.int32)
    return _tc_logits(idx, embedding_weight, out_weight)
